# baseline (speedup 1.0000x reference)
import functools
import jax
import jax.numpy as jnp
from jax.experimental import pallas as pl
from jax.experimental.pallas import tpu as pltpu

K = 3
BM = 1024
BMC = 4096


def _knn_l1_kernel(p_ref, q_ref, x_ref, y_ref, W1_ref, b1_ref,
                   h1_ref, s1_ref, ss1_ref, st_ref, *, n_src, n_steps, cx):
    i = pl.program_id(0)

    @pl.when(i == 0)
    def _():
        s1_ref[...] = jnp.zeros_like(s1_ref)
        ss1_ref[...] = jnp.zeros_like(ss1_ref)
        st_ref[1] = jnp.zeros_like(st_ref[1])

    ST_prev = st_ref[(i + 1) % 2]
    xi = jax.lax.dot_general(x_ref[0].astype(jnp.bfloat16), ST_prev,
                             dimension_numbers=(((1,), (1,)), ((), ())),
                             preferred_element_type=jnp.float32)
    W1 = W1_ref[...]
    h1 = (jax.lax.dot_general(W1[:, :cx].astype(jnp.bfloat16),
                              xi.astype(jnp.bfloat16),
                              dimension_numbers=(((1,), (0,)), ((), ())),
                              preferred_element_type=jnp.float32)
          + jax.lax.dot_general(W1[:, cx:].astype(jnp.bfloat16),
                                y_ref[0].astype(jnp.bfloat16),
                                dimension_numbers=(((1,), (0,)), ((), ())),
                                preferred_element_type=jnp.float32)
          + b1_ref[...])
    h1_ref[0] = h1.astype(jnp.bfloat16)

    live = jnp.where(i > 0, 1.0, 0.0).astype(jnp.float32)
    s1_ref[...] += live * jnp.sum(h1, axis=1, keepdims=True)
    ss1_ref[...] += live * jnp.sum(h1 * h1, axis=1, keepdims=True)

    q_blk = q_ref[0]
    pT = jnp.transpose(p_ref[0], (1, 0))

    qm2 = q_blk * (-2.0)
    pp = (pT[0:1, :] * pT[0:1, :] + pT[1:2, :] * pT[1:2, :]
          + pT[2:3, :] * pT[2:3, :])
    qq = jnp.sum(q_blk * q_blk, axis=1, keepdims=True)
    d = (qm2[:, 0:1] * pT[0:1, :] + qm2[:, 1:2] * pT[1:2, :]
         + qm2[:, 2:3] * pT[2:3, :] + pp)

    inf = jnp.float32(jnp.inf)
    t1 = jnp.min(d, axis=1, keepdims=True)
    dm1 = jnp.where(d == t1, inf, d)
    t2 = jnp.min(dm1, axis=1, keepdims=True)
    dm2 = jnp.where(dm1 == t2, inf, dm1)
    t3 = jnp.min(dm2, axis=1, keepdims=True)

    w1 = 1.0 / jnp.maximum(t1 + qq, 1e-10)
    w2 = 1.0 / jnp.maximum(t2 + qq, 1e-10)
    w3 = 1.0 / jnp.maximum(t3 + qq, 1e-10)
    wsum = w1 + w2 + w3
    wn1 = w1 / wsum
    wn2 = w2 / wsum
    wn3 = w3 / wsum

    ST = jnp.where(d == t1, wn1,
                   jnp.where(dm1 == t2, wn2,
                             jnp.where(dm1 == t3, wn3, 0.0))
                   ).astype(jnp.bfloat16)
    st_ref[i % 2] = ST


def _bn_l2_kernel(h1_ref, s1_ref, ss1_ref, g1_ref, be1_ref, W2_ref, b2_ref,
                  h2_ref, s2_ref, ss2_ref, *, count):
    b = pl.program_id(0)
    j = pl.program_id(1)

    mean = s1_ref[...] / count
    var = ss1_ref[...] / count - mean * mean
    rstd = jax.lax.rsqrt(var + 1e-5)
    scale = g1_ref[...] * rstd
    shift = be1_ref[...] - mean * scale

    h1f = h1_ref[0].astype(jnp.float32)
    h1 = jnp.maximum(h1f * scale + shift, 0.0)
    h2 = (jax.lax.dot_general(W2_ref[...].astype(jnp.bfloat16),
                              h1.astype(jnp.bfloat16),
                              dimension_numbers=(((1,), (0,)), ((), ())),
                              preferred_element_type=jnp.float32)
          + b2_ref[...])
    h2_ref[0] = h2.astype(jnp.bfloat16)

    @pl.when(jnp.logical_and(b == 0, j == 0))
    def _():
        s2_ref[...] = jnp.zeros_like(s2_ref)
        ss2_ref[...] = jnp.zeros_like(ss2_ref)

    s2_ref[...] += jnp.sum(h2, axis=1, keepdims=True)
    ss2_ref[...] += jnp.sum(h2 * h2, axis=1, keepdims=True)


def _bn_out_kernel(h2_ref, s2_ref, ss2_ref, g2_ref, be2_ref, out_ref, *,
                   count):
    mean = s2_ref[...] / count
    var = ss2_ref[...] / count - mean * mean
    rstd = jax.lax.rsqrt(var + 1e-5)
    scale = g2_ref[...] * rstd
    shift = be2_ref[...] - mean * scale
    h2f = h2_ref[0].astype(jnp.float32)
    out_ref[0] = jnp.maximum(h2f * scale + shift, 0.0)


def kernel(p, q, x, y, W1, b1, g1, be1, W2, b2, g2, be2):
    B, N, _ = p.shape
    M = q.shape[1]
    Cx = x.shape[1]
    Cy = y.shape[1]
    C1 = W1.shape[0]
    C2 = W2.shape[0]
    count = float(B * M)
    n_blocks = B * (M // BM)
    n_steps = n_blocks + 1
    jb = M // BM

    col = lambda v: v.reshape(-1, 1)

    def cur(i):
        return jnp.minimum(i, n_blocks - 1)

    def prev(i):
        return jnp.maximum(i - 1, 0)

    h1_pre, s1, ss1 = pl.pallas_call(
        functools.partial(_knn_l1_kernel, n_src=N, n_steps=n_steps, cx=Cx),
        grid=(n_steps,),
        in_specs=[
            pl.BlockSpec((1, N, 3), lambda i: (cur(i) // jb, 0, 0)),
            pl.BlockSpec((1, BM, 3), lambda i: (cur(i) // jb, cur(i) % jb, 0)),
            pl.BlockSpec((1, Cx, N), lambda i: (prev(i) // jb, 0, 0)),
            pl.BlockSpec((1, Cy, BM), lambda i: (prev(i) // jb, 0,
                                                 prev(i) % jb)),
            pl.BlockSpec((C1, Cx + Cy), lambda i: (0, 0)),
            pl.BlockSpec((C1, 1), lambda i: (0, 0)),
        ],
        out_specs=[
            pl.BlockSpec((1, C1, BM), lambda i: (prev(i) // jb, 0,
                                                 prev(i) % jb)),
            pl.BlockSpec((C1, 1), lambda i: (0, 0)),
            pl.BlockSpec((C1, 1), lambda i: (0, 0)),
        ],
        out_shape=[
            jax.ShapeDtypeStruct((B, C1, M), jnp.bfloat16),
            jax.ShapeDtypeStruct((C1, 1), jnp.float32),
            jax.ShapeDtypeStruct((C1, 1), jnp.float32),
        ],
        scratch_shapes=[pltpu.VMEM((2, BM, N), jnp.bfloat16)],
    )(p, q, x, y, W1, col(b1))

    grid_bc = (B, M // BMC)
    h2_pre, s2, ss2 = pl.pallas_call(
        functools.partial(_bn_l2_kernel, count=count),
        grid=grid_bc,
        in_specs=[
            pl.BlockSpec((1, C1, BMC), lambda b, j: (b, 0, j)),
            pl.BlockSpec((C1, 1), lambda b, j: (0, 0)),
            pl.BlockSpec((C1, 1), lambda b, j: (0, 0)),
            pl.BlockSpec((C1, 1), lambda b, j: (0, 0)),
            pl.BlockSpec((C1, 1), lambda b, j: (0, 0)),
            pl.BlockSpec((C2, C1), lambda b, j: (0, 0)),
            pl.BlockSpec((C2, 1), lambda b, j: (0, 0)),
        ],
        out_specs=[
            pl.BlockSpec((1, C2, BMC), lambda b, j: (b, 0, j)),
            pl.BlockSpec((C2, 1), lambda b, j: (0, 0)),
            pl.BlockSpec((C2, 1), lambda b, j: (0, 0)),
        ],
        out_shape=[
            jax.ShapeDtypeStruct((B, C2, M), jnp.bfloat16),
            jax.ShapeDtypeStruct((C2, 1), jnp.float32),
            jax.ShapeDtypeStruct((C2, 1), jnp.float32),
        ],
    )(h1_pre, s1, ss1, col(g1), col(be1), W2, col(b2))

    h = pl.pallas_call(
        functools.partial(_bn_out_kernel, count=count),
        grid=grid_bc,
        in_specs=[
            pl.BlockSpec((1, C2, BMC), lambda b, j: (b, 0, j)),
            pl.BlockSpec((C2, 1), lambda b, j: (0, 0)),
            pl.BlockSpec((C2, 1), lambda b, j: (0, 0)),
            pl.BlockSpec((C2, 1), lambda b, j: (0, 0)),
            pl.BlockSpec((C2, 1), lambda b, j: (0, 0)),
        ],
        out_specs=pl.BlockSpec((1, C2, BMC), lambda b, j: (b, 0, j)),
        out_shape=jax.ShapeDtypeStruct((B, C2, M), jnp.float32),
    )(h2_pre, s2, ss2, col(g2), col(be2))

    return (q, h)
